# baseline (speedup 1.0000x reference)
import jax
import jax.numpy as jnp
from jax.experimental import pallas as pl
from jax.experimental.pallas import tpu as pltpu

_S = 2
_S3 = _S ** 3


def _body(scale_ref, coords_ref, feats_ref, coords_out_ref, feats_out_ref):
    f = feats_ref[...]
    B, d = f.shape
    feats_out_ref[...] = jnp.broadcast_to(f[:, None, :], (B, _S3, d))

    c = coords_ref[...]
    s = scale_ref[0]
    b = c[:, 0:1]
    x = c[:, 1:2] * s
    y = c[:, 2:3] * s
    z = c[:, 3:4] * s
    p = jax.lax.broadcasted_iota(jnp.int32, (B, 4 * _S3), 1)
    fld = p & 3
    j = p >> 2
    out = jnp.where(
        fld == 0, b,
        jnp.where(fld == 1, x + ((j >> 2) & 1),
                  jnp.where(fld == 2, y + ((j >> 1) & 1), z + (j & 1))))
    coords_out_ref[...] = out


def kernel(coords, feats, scale):
    N, d = feats.shape
    B = 1000
    grid = (N // B,)
    scale_arr = jnp.asarray(scale, jnp.int32).reshape(1)
    coords_out, feats_out = pl.pallas_call(
        _body,
        grid=grid,
        in_specs=[
            pl.BlockSpec(memory_space=pltpu.SMEM),
            pl.BlockSpec((B, 4), lambda i: (i, 0)),
            pl.BlockSpec((B, d), lambda i: (i, 0)),
        ],
        out_specs=[
            pl.BlockSpec((B, 4 * _S3), lambda i: (i, 0)),
            pl.BlockSpec((B, _S3, d), lambda i: (i, 0, 0)),
        ],
        out_shape=[
            jax.ShapeDtypeStruct((N, 4 * _S3), jnp.int32),
            jax.ShapeDtypeStruct((N, _S3, d), jnp.float32),
        ],
    )(scale_arr, coords, feats)
    return coords_out.reshape(N * _S3, 4), feats_out.reshape(N * _S3, d)
